# baseline (speedup 1.0000x reference)
import functools

import jax
import jax.numpy as jnp
from jax import lax
from jax.experimental import pallas as pl
from jax.experimental.pallas import tpu as pltpu
from jax.experimental.pallas import tpu_sc as plsc

SEQ_SIZE = 200
EMBED_DIM = 64
BATCH = 4096


def _make_sc_gather(V, D, B):
    info = plsc.get_sparse_core_info()
    NC, NS = info.num_cores, info.num_subcores
    NW = NC * NS
    assert B % (8 * NW) == 0
    b_per_w = B // NW
    mesh = plsc.VectorSubcoreMesh(core_axis_name="c", subcore_axis_name="s")

    @functools.partial(
        pl.kernel,
        mesh=mesh,
        out_type=jax.ShapeDtypeStruct((B, D), jnp.float32),
        scratch_types=[
            pltpu.VMEM((b_per_w,), jnp.int32),
            pltpu.VMEM((b_per_w, D), jnp.float32),
            pltpu.SemaphoreType.DMA,
        ],
        compiler_params=pltpu.CompilerParams(use_tc_tiling_on_sc=False),
    )
    def gather_kernel(table_hbm, idx_hbm, out_hbm, idx_v, rows_v, sem):
        wid = lax.axis_index("s") * NC + lax.axis_index("c")
        base = wid * b_per_w
        pltpu.sync_copy(idx_hbm.at[pl.ds(base, b_per_w)], idx_v)
        pltpu.async_copy(table_hbm.at[idx_v], rows_v, sem).wait()
        pltpu.sync_copy(rows_v, out_hbm.at[pl.ds(base, b_per_w)])

    return gather_kernel


def _bcast_add_body(gath_ref, pos_ref, out_ref):
    g = gath_ref[...]
    p = pos_ref[...]
    out_ref[...] = g[:, None, :] + p[None, :, :]


def kernel(inputs, token_table, pos_table):
    V, D = token_table.shape
    B = inputs.shape[0]
    idx = inputs.reshape(B).astype(jnp.int32)

    gathered = _make_sc_gather(V, D, B)(token_table, idx)

    BB = 128
    out = pl.pallas_call(
        _bcast_add_body,
        grid=(B // BB,),
        in_specs=[
            pl.BlockSpec((BB, D), lambda i: (i, 0)),
            pl.BlockSpec((SEQ_SIZE, D), lambda i: (0, 0)),
        ],
        out_specs=pl.BlockSpec((BB, SEQ_SIZE, D), lambda i: (i, 0, 0)),
        out_shape=jax.ShapeDtypeStruct((B, SEQ_SIZE, D), jnp.float32),
    )(gathered, pos_table)
    return out
